# baseline (speedup 1.0000x reference)
import jax
import jax.numpy as jnp
from jax.experimental import pallas as pl
from jax.experimental.pallas import tpu as pltpu

_VMEM_LIMIT_BYTES = 58 * 1024 * 1024


def _pick_tile(n, target):
    best = 128
    t = 128
    while t <= min(n, target):
        if n % t == 0:
            best = t
        t *= 2
    return best


def _hop1_kernel(a_ref, x_ref, x1_ref):
    x1_ref[...] = jnp.dot(a_ref[...], x_ref[...],
                          preferred_element_type=jnp.float32
                          ).astype(jnp.bfloat16)


def _hop2_concat_kernel(a_ref, x_ref, x1_ref, o_ref):
    i = pl.program_id(0)
    ti = a_ref.shape[0]
    f = x_ref.shape[1]
    rows = pl.ds(i * ti, ti)
    o_ref[:, :f] = x_ref[rows, :]
    o_ref[:, f:2 * f] = x1_ref[rows, :].astype(jnp.float32)
    o_ref[:, 2 * f:] = jnp.dot(a_ref[...], x1_ref[...],
                               preferred_element_type=jnp.float32)


def _hop1(a, x, tile):
    n, f = x.shape
    return pl.pallas_call(
        _hop1_kernel,
        out_shape=jax.ShapeDtypeStruct((n, f), jnp.bfloat16),
        grid=(n // tile,),
        in_specs=[
            pl.BlockSpec((tile, n), lambda i: (i, 0)),
            pl.BlockSpec((n, f), lambda i: (0, 0)),
        ],
        out_specs=pl.BlockSpec((tile, f), lambda i: (i, 0)),
        compiler_params=pltpu.CompilerParams(
            dimension_semantics=("parallel",),
            vmem_limit_bytes=_VMEM_LIMIT_BYTES,
        ),
    )(a, x)


def _hop2_concat(a, x, x1, tile):
    n, f = x.shape
    return pl.pallas_call(
        _hop2_concat_kernel,
        out_shape=jax.ShapeDtypeStruct((n, 3 * f), jnp.float32),
        grid=(n // tile,),
        in_specs=[
            pl.BlockSpec((tile, n), lambda i: (i, 0)),
            pl.BlockSpec((n, f), lambda i: (0, 0)),
            pl.BlockSpec((n, f), lambda i: (0, 0)),
        ],
        out_specs=pl.BlockSpec((tile, 3 * f), lambda i: (i, 0)),
        compiler_params=pltpu.CompilerParams(
            dimension_semantics=("parallel",),
            vmem_limit_bytes=_VMEM_LIMIT_BYTES,
        ),
    )(a, x, x1)


def kernel(x, a):
    n, _ = x.shape
    x1 = _hop1(a, x, _pick_tile(n, 256))
    return _hop2_concat(a, x, x1, _pick_tile(n, 512))


# speedup vs baseline: 1.0742x; 1.0742x over previous
import jax
import jax.numpy as jnp
from jax.experimental import pallas as pl
from jax.experimental.pallas import tpu as pltpu

_VMEM_LIMIT_BYTES = 58 * 1024 * 1024


def _pick_tile(n, target):
    best = 128
    t = 128
    while t <= min(n, target):
        if n % t == 0:
            best = t
        t *= 2
    return best


def _hop1_kernel(a_ref, x_ref, x1_ref):
    x1_ref[...] = jnp.dot(a_ref[...], x_ref[...],
                          preferred_element_type=jnp.float32
                          ).astype(jnp.bfloat16)


def _hop2_concat_kernel(a_ref, x_ref, x1_ref, o_ref):
    i = pl.program_id(0)
    ti = a_ref.shape[0]
    f = x_ref.shape[1]
    rows = pl.ds(i * ti, ti)
    o_ref[:, :f] = x_ref[rows, :]
    o_ref[:, f:2 * f] = x1_ref[rows, :].astype(jnp.float32)
    o_ref[:, 2 * f:] = jnp.dot(a_ref[...], x1_ref[...],
                               preferred_element_type=jnp.float32)


def _hop1(a, x, tile):
    n, f = x.shape
    return pl.pallas_call(
        _hop1_kernel,
        out_shape=jax.ShapeDtypeStruct((n, f), jnp.bfloat16),
        grid=(n // tile,),
        in_specs=[
            pl.BlockSpec((tile, n), lambda i: (i, 0)),
            pl.BlockSpec((n, f), lambda i: (0, 0)),
        ],
        out_specs=pl.BlockSpec((tile, f), lambda i: (i, 0)),
        compiler_params=pltpu.CompilerParams(
            dimension_semantics=("parallel",),
            vmem_limit_bytes=_VMEM_LIMIT_BYTES,
        ),
    )(a, x)


def _hop2_concat(a, x, x1, tile):
    n, f = x.shape
    return pl.pallas_call(
        _hop2_concat_kernel,
        out_shape=jax.ShapeDtypeStruct((n, 3 * f), jnp.float32),
        grid=(n // tile,),
        in_specs=[
            pl.BlockSpec((tile, n), lambda i: (i, 0)),
            pl.BlockSpec((n, f), lambda i: (0, 0)),
            pl.BlockSpec((n, f), lambda i: (0, 0)),
        ],
        out_specs=pl.BlockSpec((tile, 3 * f), lambda i: (i, 0)),
        compiler_params=pltpu.CompilerParams(
            dimension_semantics=("parallel",),
            vmem_limit_bytes=_VMEM_LIMIT_BYTES,
        ),
    )(a, x, x1)


def kernel(x, a):
    n, _ = x.shape
    tile = _pick_tile(n, 512)
    x1 = _hop1(a, x, tile)
    return _hop2_concat(a, x, x1, tile)


# speedup vs baseline: 1.3519x; 1.2585x over previous
import jax
import jax.numpy as jnp
from jax.experimental import pallas as pl
from jax.experimental.pallas import tpu as pltpu

_VMEM_LIMIT_BYTES = 58 * 1024 * 1024
_SLAB = 256


def _fused_kernel(a_ref, x_ref, o_ref, abf_ref, x1b_ref):
    p = pl.program_id(0)
    i = pl.program_id(1)
    ns = a_ref.shape[0]
    f = x_ref.shape[1]
    rows = pl.ds(i * ns, ns)

    @pl.when(p == 0)
    def _():
        aslab = a_ref[...]
        x1 = jnp.dot(aslab, x_ref[...], preferred_element_type=jnp.float32)
        x1b_ref[rows, :] = x1.astype(jnp.bfloat16)
        abf_ref[rows, :] = aslab.astype(jnp.bfloat16)

    @pl.when(p == 1)
    def _():
        o_ref[:, :f] = x_ref[rows, :]
        o_ref[:, f:2 * f] = x1b_ref[rows, :].astype(jnp.float32)
        o_ref[:, 2 * f:] = jnp.dot(abf_ref[rows, :], x1b_ref[...],
                                   preferred_element_type=jnp.float32)


def kernel(x, a):
    n, f = x.shape
    slab = _SLAB if n % _SLAB == 0 else n
    nblk = n // slab
    return pl.pallas_call(
        _fused_kernel,
        out_shape=jax.ShapeDtypeStruct((n, 3 * f), jnp.float32),
        grid=(2, nblk),
        in_specs=[
            pl.BlockSpec((slab, n),
                         lambda p, i: (jnp.where(p == 0, i, nblk - 1), 0)),
            pl.BlockSpec((n, f), lambda p, i: (0, 0)),
        ],
        out_specs=pl.BlockSpec(
            (slab, 3 * f), lambda p, i: (jnp.where(p == 0, 0, i), 0)),
        scratch_shapes=[
            pltpu.VMEM((n, n), jnp.bfloat16),
            pltpu.VMEM((n, f), jnp.bfloat16),
        ],
        compiler_params=pltpu.CompilerParams(
            dimension_semantics=("arbitrary", "arbitrary"),
            vmem_limit_bytes=_VMEM_LIMIT_BYTES,
        ),
    )(a, x)


# speedup vs baseline: 1.5404x; 1.1394x over previous
import jax
import jax.numpy as jnp
from jax.experimental import pallas as pl
from jax.experimental.pallas import tpu as pltpu

_VMEM_LIMIT_BYTES = 58 * 1024 * 1024
_SLAB = 512


def _fused_kernel(a_ref, x_ref, o_ref, abf_ref, x1b_ref):
    p = pl.program_id(0)
    i = pl.program_id(1)
    ns = a_ref.shape[0]
    f = x_ref.shape[1]
    rows = pl.ds(i * ns, ns)

    @pl.when(p == 0)
    def _():
        aslab = a_ref[...]
        x1 = jnp.dot(aslab, x_ref[...], preferred_element_type=jnp.float32)
        x1b_ref[rows, :] = x1.astype(jnp.bfloat16)
        abf_ref[rows, :] = aslab.astype(jnp.bfloat16)

    @pl.when(p == 1)
    def _():
        o_ref[:, :f] = x_ref[rows, :]
        o_ref[:, f:2 * f] = x1b_ref[rows, :].astype(jnp.float32)
        o_ref[:, 2 * f:] = jnp.dot(abf_ref[rows, :], x1b_ref[...],
                                   preferred_element_type=jnp.float32)


def kernel(x, a):
    n, f = x.shape
    slab = _SLAB if n % _SLAB == 0 else n
    nblk = n // slab
    return pl.pallas_call(
        _fused_kernel,
        out_shape=jax.ShapeDtypeStruct((n, 3 * f), jnp.float32),
        grid=(2, nblk),
        in_specs=[
            pl.BlockSpec((slab, n),
                         lambda p, i: (jnp.where(p == 0, i, nblk - 1), 0)),
            pl.BlockSpec((n, f), lambda p, i: (0, 0)),
        ],
        out_specs=pl.BlockSpec(
            (slab, 3 * f), lambda p, i: (jnp.where(p == 0, 0, i), 0)),
        scratch_shapes=[
            pltpu.VMEM((n, n), jnp.bfloat16),
            pltpu.VMEM((n, f), jnp.bfloat16),
        ],
        compiler_params=pltpu.CompilerParams(
            dimension_semantics=("arbitrary", "arbitrary"),
            vmem_limit_bytes=_VMEM_LIMIT_BYTES,
        ),
    )(a, x)


# speedup vs baseline: 1.5786x; 1.0248x over previous
import jax
import jax.numpy as jnp
from jax.experimental import pallas as pl
from jax.experimental.pallas import tpu as pltpu

_VMEM_LIMIT_BYTES = 64 * 1024 * 1024
_SLAB = 512


def _pick_slab(n):
    t = _SLAB
    while t >= 128:
        if n % t == 0 and (n // t) % 2 == 0:
            return t
        t //= 2
    return n


def _fused_kernel(a_ref, x_ref, o_ref, abf_ref, x1b_ref, x2l_ref):
    p = pl.program_id(0)
    i = pl.program_id(1)
    ns = a_ref.shape[0]
    n = x_ref.shape[0]
    f = x_ref.shape[1]
    h = (n // ns) // 2
    kh = n // 2
    rows = pl.ds(i * ns, ns)

    @pl.when(p == 0)
    def _():
        aslab = a_ref[...]
        x1 = jnp.dot(aslab, x_ref[...], preferred_element_type=jnp.float32)
        x1b_ref[rows, :] = x1.astype(jnp.bfloat16)
        abf_ref[rows, :] = aslab.astype(jnp.bfloat16)

        @pl.when(i >= h)
        def _():
            for s in (i - h, i):
                rs = pl.ds(s * ns, ns)
                x2l_ref[rs, :] = jnp.dot(
                    abf_ref[rs, :kh], x1b_ref[:kh, :],
                    preferred_element_type=jnp.float32)

    @pl.when(p == 1)
    def _():
        right = jnp.dot(abf_ref[rows, kh:], x1b_ref[kh:, :],
                        preferred_element_type=jnp.float32)
        o_ref[:, :f] = x_ref[rows, :]
        o_ref[:, f:2 * f] = x1b_ref[rows, :].astype(jnp.float32)
        o_ref[:, 2 * f:] = x2l_ref[rows, :] + right


def kernel(x, a):
    n, f = x.shape
    slab = _pick_slab(n)
    nblk = n // slab
    return pl.pallas_call(
        _fused_kernel,
        out_shape=jax.ShapeDtypeStruct((n, 3 * f), jnp.float32),
        grid=(2, nblk),
        in_specs=[
            pl.BlockSpec((slab, n),
                         lambda p, i: (jnp.where(p == 0, i, nblk - 1), 0)),
            pl.BlockSpec((n, f), lambda p, i: (0, 0)),
        ],
        out_specs=pl.BlockSpec(
            (slab, 3 * f), lambda p, i: (jnp.where(p == 0, 0, i), 0)),
        scratch_shapes=[
            pltpu.VMEM((n, n), jnp.bfloat16),
            pltpu.VMEM((n, f), jnp.bfloat16),
            pltpu.VMEM((n, f), jnp.float32),
        ],
        compiler_params=pltpu.CompilerParams(
            dimension_semantics=("arbitrary", "arbitrary"),
            vmem_limit_bytes=_VMEM_LIMIT_BYTES,
        ),
    )(a, x)


# speedup vs baseline: 1.5800x; 1.0009x over previous
import jax
import jax.numpy as jnp
from jax.experimental import pallas as pl
from jax.experimental.pallas import tpu as pltpu

_VMEM_LIMIT_BYTES = 64 * 1024 * 1024
_SLAB = 512


def _pick_slab(n):
    t = _SLAB
    while t >= 128:
        if n % t == 0 and (n // t) % 2 == 0:
            return t
        t //= 2
    return n


def _fused_kernel(a_ref, x_ref, o_ref, abf_ref, x1b_ref, x2l_ref):
    p = pl.program_id(0)
    i = pl.program_id(1)
    ns = a_ref.shape[0]
    n = x_ref.shape[0]
    f = x_ref.shape[1]
    h = (n // ns) // 2
    kh = n // 4
    rows = pl.ds(i * ns, ns)

    @pl.when(p == 0)
    def _():
        aslab = a_ref[...]
        x1 = jnp.dot(aslab, x_ref[...], preferred_element_type=jnp.float32)
        x1b_ref[rows, :] = x1.astype(jnp.bfloat16)
        abf_ref[rows, :] = aslab.astype(jnp.bfloat16)

        @pl.when(i >= h)
        def _():
            for s in (i - h, i):
                rs = pl.ds(s * ns, ns)
                x2l_ref[rs, :] = jnp.dot(
                    abf_ref[rs, :kh], x1b_ref[:kh, :],
                    preferred_element_type=jnp.float32)

    @pl.when(p == 1)
    def _():
        right = jnp.dot(abf_ref[rows, kh:], x1b_ref[kh:, :],
                        preferred_element_type=jnp.float32)
        o_ref[:, :f] = x_ref[rows, :]
        o_ref[:, f:2 * f] = x1b_ref[rows, :].astype(jnp.float32)
        o_ref[:, 2 * f:] = x2l_ref[rows, :] + right


def kernel(x, a):
    n, f = x.shape
    slab = _pick_slab(n)
    nblk = n // slab
    return pl.pallas_call(
        _fused_kernel,
        out_shape=jax.ShapeDtypeStruct((n, 3 * f), jnp.float32),
        grid=(2, nblk),
        in_specs=[
            pl.BlockSpec((slab, n),
                         lambda p, i: (jnp.where(p == 0, i, nblk - 1), 0)),
            pl.BlockSpec((n, f), lambda p, i: (0, 0)),
        ],
        out_specs=pl.BlockSpec(
            (slab, 3 * f), lambda p, i: (jnp.where(p == 0, 0, i), 0)),
        scratch_shapes=[
            pltpu.VMEM((n, n), jnp.bfloat16),
            pltpu.VMEM((n, f), jnp.bfloat16),
            pltpu.VMEM((n, f), jnp.float32),
        ],
        compiler_params=pltpu.CompilerParams(
            dimension_semantics=("arbitrary", "arbitrary"),
            vmem_limit_bytes=_VMEM_LIMIT_BYTES,
        ),
    )(a, x)


# speedup vs baseline: 1.5861x; 1.0039x over previous
import jax
import jax.numpy as jnp
from jax.experimental import pallas as pl
from jax.experimental.pallas import tpu as pltpu

_VMEM_LIMIT_BYTES = 64 * 1024 * 1024
_SLAB = 512


def _pick_slab(n):
    t = _SLAB
    while t >= 128:
        if n % t == 0 and (n // t) % 2 == 0:
            return t
        t //= 2
    return n


def _fused_kernel(a_ref, x_ref, o_ref, abf_ref, x1b_ref, x2l_ref):
    p = pl.program_id(0)
    i = pl.program_id(1)
    ns = a_ref.shape[0]
    n = x_ref.shape[0]
    f = x_ref.shape[1]
    h = (n // ns) // 2
    kh = n // 2
    rows = pl.ds(i * ns, ns)

    @pl.when(p == 0)
    def _():
        aslab = a_ref[...]
        x1 = jnp.dot(aslab, x_ref[...], preferred_element_type=jnp.float32)
        x1b_ref[rows, :] = x1.astype(jnp.bfloat16)
        abf_ref[rows, :] = aslab.astype(jnp.bfloat16)

        @pl.when(i >= h)
        def _():
            for s in (i - h, i):
                rs = pl.ds(s * ns, ns)
                x2l_ref[rs, :] = jnp.dot(
                    abf_ref[rs, :kh], x1b_ref[:kh, :],
                    preferred_element_type=jnp.float32)

    @pl.when(p == 1)
    def _():
        right = jnp.dot(abf_ref[rows, kh:], x1b_ref[kh:, :],
                        preferred_element_type=jnp.float32)
        o_ref[:, :f] = x_ref[rows, :]
        o_ref[:, f:2 * f] = x1b_ref[rows, :].astype(jnp.float32)
        o_ref[:, 2 * f:] = x2l_ref[rows, :] + right


def kernel(x, a):
    n, f = x.shape
    slab = _pick_slab(n)
    nblk = n // slab
    return pl.pallas_call(
        _fused_kernel,
        out_shape=jax.ShapeDtypeStruct((n, 3 * f), jnp.float32),
        grid=(2, nblk),
        in_specs=[
            pl.BlockSpec((slab, n),
                         lambda p, i: (jnp.where(p == 0, i, nblk - 1), 0)),
            pl.BlockSpec((n, f), lambda p, i: (0, 0)),
        ],
        out_specs=pl.BlockSpec(
            (slab, 3 * f), lambda p, i: (jnp.where(p == 0, 0, i), 0)),
        scratch_shapes=[
            pltpu.VMEM((n, n), jnp.bfloat16),
            pltpu.VMEM((n, f), jnp.bfloat16),
            pltpu.VMEM((n, f), jnp.float32),
        ],
        compiler_params=pltpu.CompilerParams(
            dimension_semantics=("arbitrary", "arbitrary"),
            vmem_limit_bytes=_VMEM_LIMIT_BYTES,
        ),
    )(a, x)
